# baseline (device time: 48980 ns/iter reference)
import jax
import jax.numpy as jnp
from jax import lax
from jax.experimental import pallas as pl
from jax.experimental.pallas import tpu as pltpu

N_DEV = 4
BLOCK = 64
DH = 64
B = 2
SQ = 512
DM = 768
HALF = DM // 2
H = 8
NCHUNK = 4
R = B * SQ // NCHUNK


def _fused_body(
    x_ref, wq_ref, k_ref, v_ref, wo_ref, out_ref,
    pa, pb, acc_a, acc_b, ra1, rb1, ra2, rb2,
    send_sems, recv_sems,
):
    my = lax.axis_index("i")
    p_y = my ^ 1
    p_x = 3 - my
    bf = jnp.bfloat16
    f32 = jnp.float32

    barrier = pltpu.get_barrier_semaphore()
    for nbr in [p_y, p_x]:
        pl.semaphore_signal(
            barrier, inc=1, device_id=(nbr,), device_id_type=pl.DeviceIdType.MESH
        )

    qi = lax.broadcasted_iota(jnp.int32, (SQ, SQ), 0) // BLOCK
    kj = lax.broadcasted_iota(jnp.int32, (SQ, SQ), 1) // BLOCK
    mask = (qi == kj) | (kj == 0) | ((qi + kj) % 3 == 0)
    bias = jnp.where(mask, 0.0, -1e9).astype(f32)

    def compute_chunk(c):
        b, o = c // 2, (c % 2) * R
        r0 = c * R
        q = jnp.dot(
            x_ref[b, o:o + R, :], wq_ref[...], preferred_element_type=f32
        )
        bias_c = bias[o:o + R, :]
        ctx_heads = []
        for h in range(H):
            qh = q[:, h * DH:(h + 1) * DH].astype(bf)
            kh = k_ref[b, :, h, :]
            s = jnp.dot(qh, kh.T, preferred_element_type=f32) * 0.125 + bias_c
            e = jnp.exp(s)
            w = (e * (1.0 / e.sum(axis=-1, keepdims=True))).astype(bf)
            ctx_heads.append(
                jnp.dot(w, v_ref[b, :, h, :], preferred_element_type=f32)
            )
        ctx = jnp.concatenate(ctx_heads, axis=1).astype(bf)
        part = jnp.dot(ctx, wo_ref[...], preferred_element_type=f32)
        pa[r0:r0 + R, :] = part[:, :HALF].astype(bf)
        pb[r0:r0 + R, :] = part[:, HALF:].astype(bf)

    def phase1(c):
        r0 = c * R
        ca = pltpu.make_async_remote_copy(
            src_ref=pa.at[r0:r0 + R, :], dst_ref=ra1.at[r0:r0 + R, :],
            send_sem=send_sems.at[4 * c + 0], recv_sem=recv_sems.at[4 * c + 0],
            device_id=(p_y,), device_id_type=pl.DeviceIdType.MESH,
        )
        cb = pltpu.make_async_remote_copy(
            src_ref=pb.at[r0:r0 + R, :], dst_ref=rb1.at[r0:r0 + R, :],
            send_sem=send_sems.at[4 * c + 1], recv_sem=recv_sems.at[4 * c + 1],
            device_id=(p_x,), device_id_type=pl.DeviceIdType.MESH,
        )
        ca.start()
        cb.start()
        return ca, cb

    def phase2(c):
        r0 = c * R
        ca = pltpu.make_async_remote_copy(
            src_ref=acc_a.at[r0:r0 + R, :], dst_ref=ra2.at[r0:r0 + R, :],
            send_sem=send_sems.at[4 * c + 2], recv_sem=recv_sems.at[4 * c + 2],
            device_id=(p_x,), device_id_type=pl.DeviceIdType.MESH,
        )
        cb = pltpu.make_async_remote_copy(
            src_ref=acc_b.at[r0:r0 + R, :], dst_ref=rb2.at[r0:r0 + R, :],
            send_sem=send_sems.at[4 * c + 3], recv_sem=recv_sems.at[4 * c + 3],
            device_id=(p_y,), device_id_type=pl.DeviceIdType.MESH,
        )
        ca.start()
        cb.start()
        return ca, cb

    def wait2(pair):
        pair[0].wait()
        pair[1].wait()

    def add1(c):
        sl = pl.ds(c * R, R)
        acc_a[sl, :] = pa[sl, :] + ra1[sl, :]
        acc_b[sl, :] = pb[sl, :] + rb1[sl, :]

    def emit(c):
        sl = pl.ds(c * R, R)
        out_ref[sl, :HALF] = acc_a[sl, :].astype(f32) + ra2[sl, :].astype(f32)
        out_ref[sl, HALF:] = acc_b[sl, :].astype(f32) + rb2[sl, :].astype(f32)

    p1 = [None] * NCHUNK
    p2 = [None] * NCHUNK

    compute_chunk(0)
    pl.semaphore_wait(barrier, 2)
    p1[0] = phase1(0)
    compute_chunk(1)
    p1[1] = phase1(1)
    wait2(p1[0]); add1(0); p2[0] = phase2(0)
    compute_chunk(2)
    p1[2] = phase1(2)
    wait2(p1[1]); add1(1); p2[1] = phase2(1)
    compute_chunk(3)
    p1[3] = phase1(3)
    wait2(p1[2]); add1(2); p2[2] = phase2(2)
    wait2(p2[0]); emit(0)
    wait2(p1[3]); add1(3); p2[3] = phase2(3)
    wait2(p2[1]); emit(1)
    wait2(p2[2]); emit(2)
    wait2(p2[3]); emit(3)


def kernel(x, Wq, K_ext, V_ext, Wo):
    bf = jnp.bfloat16
    my = lax.axis_index("i")
    K = lax.dynamic_slice_in_dim(K_ext, my * H, H, axis=2).astype(bf)
    V = lax.dynamic_slice_in_dim(V_ext, my * H, H, axis=2).astype(bf)

    m = B * SQ
    half_buf = pltpu.VMEM((m, HALF), bf)
    out = pl.pallas_call(
        _fused_body,
        out_shape=jax.ShapeDtypeStruct((m, DM), jnp.float32),
        in_specs=[pl.BlockSpec(memory_space=pltpu.VMEM)] * 5,
        out_specs=pl.BlockSpec(memory_space=pltpu.VMEM),
        scratch_shapes=[
            half_buf, half_buf,
            half_buf, half_buf,
            half_buf, half_buf, half_buf, half_buf,
            pltpu.SemaphoreType.DMA((4 * NCHUNK,)),
            pltpu.SemaphoreType.DMA((4 * NCHUNK,)),
        ],
        compiler_params=pltpu.CompilerParams(collective_id=0),
    )(x.astype(bf), Wq.astype(bf), K, V, Wo.astype(bf))
    return out.reshape(B, SQ, DM)


# device time: 25521 ns/iter; 1.9192x vs baseline; 1.9192x over previous
import os

import jax
import jax.numpy as jnp
from jax import lax
from jax.experimental import pallas as pl
from jax.experimental.pallas import tpu as pltpu

N_DEV = 4
BLOCK = 64
DH = 64
B = 2
SQ = 512
DM = 768
HALF = DM // 2
H = 8
NCHUNK = 2
R = B * SQ // NCHUNK

_COMM = os.environ.get("KERNEL_NO_COMM") != "1"


def _fused_body(
    x_ref, wq_ref, k_ref, v_ref, wo_ref, out_ref,
    pa, pb, acc_a, acc_b, ra1, rb1, ra2, rb2,
    send_sems, recv_sems,
):
    my = lax.axis_index("i")
    p_y = my ^ 1
    p_x = 3 - my
    bf = jnp.bfloat16
    f32 = jnp.float32

    barrier = pltpu.get_barrier_semaphore()
    for nbr in [p_y, p_x]:
        pl.semaphore_signal(
            barrier, inc=1, device_id=(nbr,), device_id_type=pl.DeviceIdType.MESH
        )

    qi = lax.broadcasted_iota(jnp.int32, (SQ, SQ), 0) // BLOCK
    kj = lax.broadcasted_iota(jnp.int32, (SQ, SQ), 1) // BLOCK
    mask = (qi == kj) | (kj == 0) | ((qi + kj) % 3 == 0)
    bias = jnp.where(mask, 0.0, -1e9).astype(f32)

    def compute_chunk(c):
        r0 = c * R
        b, o = r0 // SQ, r0 % SQ
        q = jnp.dot(
            x_ref[b, o:o + R, :], wq_ref[...], preferred_element_type=f32
        )
        bias_c = bias[o:o + R, :]
        ctx_heads = []
        for h in range(H):
            qh = q[:, h * DH:(h + 1) * DH].astype(bf)
            kh = k_ref[b, :, h, :]
            s = jnp.dot(qh, kh.T, preferred_element_type=f32) * 0.125 + bias_c
            e = jnp.exp(s)
            w = (e * (1.0 / e.sum(axis=-1, keepdims=True))).astype(bf)
            ctx_heads.append(
                jnp.dot(w, v_ref[b, :, h, :], preferred_element_type=f32)
            )
        ctx = jnp.concatenate(ctx_heads, axis=1).astype(bf)
        part = jnp.dot(ctx, wo_ref[...], preferred_element_type=f32)
        pa[r0:r0 + R, :] = part[:, :HALF].astype(bf)
        pb[r0:r0 + R, :] = part[:, HALF:].astype(bf)

    def phase1(c):
        r0 = c * R
        ca = pltpu.make_async_remote_copy(
            src_ref=pa.at[r0:r0 + R, :], dst_ref=ra1.at[r0:r0 + R, :],
            send_sem=send_sems.at[4 * c + 0], recv_sem=recv_sems.at[4 * c + 0],
            device_id=(p_y,), device_id_type=pl.DeviceIdType.MESH,
        )
        cb = pltpu.make_async_remote_copy(
            src_ref=pb.at[r0:r0 + R, :], dst_ref=rb1.at[r0:r0 + R, :],
            send_sem=send_sems.at[4 * c + 1], recv_sem=recv_sems.at[4 * c + 1],
            device_id=(p_x,), device_id_type=pl.DeviceIdType.MESH,
        )
        ca.start()
        cb.start()
        return ca, cb

    def phase2(c):
        r0 = c * R
        ca = pltpu.make_async_remote_copy(
            src_ref=acc_a.at[r0:r0 + R, :], dst_ref=ra2.at[r0:r0 + R, :],
            send_sem=send_sems.at[4 * c + 2], recv_sem=recv_sems.at[4 * c + 2],
            device_id=(p_x,), device_id_type=pl.DeviceIdType.MESH,
        )
        cb = pltpu.make_async_remote_copy(
            src_ref=acc_b.at[r0:r0 + R, :], dst_ref=rb2.at[r0:r0 + R, :],
            send_sem=send_sems.at[4 * c + 3], recv_sem=recv_sems.at[4 * c + 3],
            device_id=(p_y,), device_id_type=pl.DeviceIdType.MESH,
        )
        ca.start()
        cb.start()
        return ca, cb

    def wait2(pair):
        pair[0].wait()
        pair[1].wait()

    def add1(c):
        sl = pl.ds(c * R, R)
        acc_a[sl, :] = pa[sl, :] + ra1[sl, :]
        acc_b[sl, :] = pb[sl, :] + rb1[sl, :]

    def emit(c):
        sl = pl.ds(c * R, R)
        out_ref[sl, :HALF] = acc_a[sl, :].astype(f32) + ra2[sl, :].astype(f32)
        out_ref[sl, HALF:] = acc_b[sl, :].astype(f32) + rb2[sl, :].astype(f32)

    if not _COMM:
        for c in range(NCHUNK):
            compute_chunk(c)
        for c in range(NCHUNK):
            sl = pl.ds(c * R, R)
            out_ref[sl, :HALF] = pa[sl, :].astype(f32)
            out_ref[sl, HALF:] = pb[sl, :].astype(f32)
        return

    p1 = [None] * NCHUNK
    p2 = [None] * NCHUNK
    for c in range(NCHUNK):
        compute_chunk(c)
        if c == 0:
            pl.semaphore_wait(barrier, 2)
        p1[c] = phase1(c)
        if c >= 1:
            wait2(p1[c - 1]); add1(c - 1); p2[c - 1] = phase2(c - 1)
    wait2(p1[NCHUNK - 1]); add1(NCHUNK - 1); p2[NCHUNK - 1] = phase2(NCHUNK - 1)
    for c in range(NCHUNK):
        wait2(p2[c]); emit(c)


def kernel(x, Wq, K_ext, V_ext, Wo):
    bf = jnp.bfloat16
    my = lax.axis_index("i")
    K = lax.dynamic_slice_in_dim(K_ext, my * H, H, axis=2).astype(bf)
    V = lax.dynamic_slice_in_dim(V_ext, my * H, H, axis=2).astype(bf)

    m = B * SQ
    half_buf = pltpu.VMEM((m, HALF), bf)
    out = pl.pallas_call(
        _fused_body,
        out_shape=jax.ShapeDtypeStruct((m, DM), jnp.float32),
        in_specs=[pl.BlockSpec(memory_space=pltpu.VMEM)] * 5,
        out_specs=pl.BlockSpec(memory_space=pltpu.VMEM),
        scratch_shapes=[
            half_buf, half_buf,
            half_buf, half_buf,
            half_buf, half_buf, half_buf, half_buf,
            pltpu.SemaphoreType.DMA((4 * NCHUNK,)),
            pltpu.SemaphoreType.DMA((4 * NCHUNK,)),
        ],
        compiler_params=pltpu.CompilerParams(collective_id=0),
    )(x.astype(bf), Wq.astype(bf), K, V, Wo.astype(bf))
    return out.reshape(B, SQ, DM)


# device time: 19425 ns/iter; 2.5215x vs baseline; 1.3138x over previous
import os

import jax
import jax.numpy as jnp
from jax import lax
from jax.experimental import pallas as pl
from jax.experimental.pallas import tpu as pltpu

N_DEV = 4
BLOCK = 64
DH = 64
B = 2
SQ = 512
DM = 768
HALF = DM // 2
H = 8
NCHUNK = 4
R = B * SQ // NCHUNK

_COMM = os.environ.get("KERNEL_NO_COMM") != "1"


def _fused_body(
    x_ref, wq_ref, k_ref, v_ref, wo_ref, out_ref,
    pa, pb, acc_a, acc_b, ra1, rb1, ra2, rb2,
    send_sems, recv_sems,
):
    my = lax.axis_index("i")
    p_y = my ^ 1
    p_x = 3 - my
    bf = jnp.bfloat16
    f32 = jnp.float32

    barrier = pltpu.get_barrier_semaphore()
    for nbr in [p_y, p_x]:
        pl.semaphore_signal(
            barrier, inc=1, device_id=(nbr,), device_id_type=pl.DeviceIdType.MESH
        )

    qi = lax.broadcasted_iota(jnp.int32, (SQ, SQ), 0) // BLOCK
    kj = lax.broadcasted_iota(jnp.int32, (SQ, SQ), 1) // BLOCK
    mask = (qi == kj) | (kj == 0) | ((qi + kj) % 3 == 0)
    bias = jnp.where(mask, 0.0, -1e9).astype(f32)

    wqv = wq_ref[...].astype(bf)
    wov = wo_ref[...].astype(bf)

    def compute_chunk(c):
        r0 = c * R
        b, o = r0 // SQ, r0 % SQ
        q = jnp.dot(
            x_ref[b, o:o + R, :].astype(bf), wqv, preferred_element_type=f32
        )
        bias_c = bias[o:o + R, :]
        ctx_heads = []
        for h in range(H):
            qh = q[:, h * DH:(h + 1) * DH].astype(bf)
            kht = k_ref[b, h]
            s = jnp.dot(qh, kht, preferred_element_type=f32) * 0.125 + bias_c
            e = jnp.exp(s)
            w = (e * (1.0 / e.sum(axis=-1, keepdims=True))).astype(bf)
            ctx_heads.append(
                jnp.dot(w, v_ref[b, h], preferred_element_type=f32)
            )
        ctx = jnp.concatenate(ctx_heads, axis=1).astype(bf)
        part = jnp.dot(ctx, wov, preferred_element_type=f32)
        pa[r0:r0 + R, :] = part[:, :HALF].astype(bf)
        pb[r0:r0 + R, :] = part[:, HALF:].astype(bf)

    def phase1(c):
        r0 = c * R
        ca = pltpu.make_async_remote_copy(
            src_ref=pa.at[r0:r0 + R, :], dst_ref=ra1.at[r0:r0 + R, :],
            send_sem=send_sems.at[4 * c + 0], recv_sem=recv_sems.at[4 * c + 0],
            device_id=(p_y,), device_id_type=pl.DeviceIdType.MESH,
        )
        cb = pltpu.make_async_remote_copy(
            src_ref=pb.at[r0:r0 + R, :], dst_ref=rb1.at[r0:r0 + R, :],
            send_sem=send_sems.at[4 * c + 1], recv_sem=recv_sems.at[4 * c + 1],
            device_id=(p_x,), device_id_type=pl.DeviceIdType.MESH,
        )
        ca.start()
        cb.start()
        return ca, cb

    def phase2(c):
        r0 = c * R
        ca = pltpu.make_async_remote_copy(
            src_ref=acc_a.at[r0:r0 + R, :], dst_ref=ra2.at[r0:r0 + R, :],
            send_sem=send_sems.at[4 * c + 2], recv_sem=recv_sems.at[4 * c + 2],
            device_id=(p_x,), device_id_type=pl.DeviceIdType.MESH,
        )
        cb = pltpu.make_async_remote_copy(
            src_ref=acc_b.at[r0:r0 + R, :], dst_ref=rb2.at[r0:r0 + R, :],
            send_sem=send_sems.at[4 * c + 3], recv_sem=recv_sems.at[4 * c + 3],
            device_id=(p_y,), device_id_type=pl.DeviceIdType.MESH,
        )
        ca.start()
        cb.start()
        return ca, cb

    def wait2(pair):
        pair[0].wait()
        pair[1].wait()

    def add1(c):
        sl = pl.ds(c * R, R)
        acc_a[sl, :] = pa[sl, :] + ra1[sl, :]
        acc_b[sl, :] = pb[sl, :] + rb1[sl, :]

    def emit(c):
        sl = pl.ds(c * R, R)
        out_ref[sl, :HALF] = acc_a[sl, :].astype(f32) + ra2[sl, :].astype(f32)
        out_ref[sl, HALF:] = acc_b[sl, :].astype(f32) + rb2[sl, :].astype(f32)

    if not _COMM:
        for c in range(NCHUNK):
            compute_chunk(c)
        for c in range(NCHUNK):
            sl = pl.ds(c * R, R)
            out_ref[sl, :HALF] = pa[sl, :].astype(f32)
            out_ref[sl, HALF:] = pb[sl, :].astype(f32)
        return

    p1 = [None] * NCHUNK
    p2 = [None] * NCHUNK
    for c in range(NCHUNK):
        compute_chunk(c)
        if c == 0:
            pl.semaphore_wait(barrier, 2)
        p1[c] = phase1(c)
        if c >= 1:
            wait2(p1[c - 1]); add1(c - 1); p2[c - 1] = phase2(c - 1)
    wait2(p1[NCHUNK - 1]); add1(NCHUNK - 1); p2[NCHUNK - 1] = phase2(NCHUNK - 1)
    for c in range(NCHUNK):
        wait2(p2[c]); emit(c)


def kernel(x, Wq, K_ext, V_ext, Wo):
    bf = jnp.bfloat16
    my = lax.axis_index("i")
    K = lax.dynamic_slice_in_dim(K_ext, my * H, H, axis=2)
    V = lax.dynamic_slice_in_dim(V_ext, my * H, H, axis=2)
    Kt = jnp.transpose(K, (0, 2, 3, 1)).astype(bf)
    Vt = jnp.transpose(V, (0, 2, 1, 3)).astype(bf)

    m = B * SQ
    half_buf = pltpu.VMEM((m, HALF), bf)
    out = pl.pallas_call(
        _fused_body,
        out_shape=jax.ShapeDtypeStruct((m, DM), jnp.float32),
        in_specs=[pl.BlockSpec(memory_space=pltpu.VMEM)] * 5,
        out_specs=pl.BlockSpec(memory_space=pltpu.VMEM),
        scratch_shapes=[
            half_buf, half_buf,
            half_buf, half_buf,
            half_buf, half_buf, half_buf, half_buf,
            pltpu.SemaphoreType.DMA((4 * NCHUNK,)),
            pltpu.SemaphoreType.DMA((4 * NCHUNK,)),
        ],
        compiler_params=pltpu.CompilerParams(collective_id=0),
    )(x, Wq, Kt, Vt, Wo)
    return out.reshape(B, SQ, DM)
